# baseline (speedup 1.0000x reference)
import jax
import jax.numpy as jnp
from jax import lax
from jax.experimental import pallas as pl
from jax.experimental.pallas import tpu as pltpu
from jax.experimental.pallas import tpu_sc as plsc

N = 100000
D = 128
H = 8
S = 512
BN = 4096
NB = -(-N // BN)
NP = NB * BN

NC = 2
NS = 16
NW = NC * NS
C = 128
NBLK = N // C
TAIL = N - NBLK * C
EXTRA = NBLK - (NBLK // NW) * NW
NBLK_BASE = NBLK // NW
L = 16


def _pass_a(x_ref, b3_ref, w_ref, bias_ref, e_ref, rd_ref, denom_ref):
    i = pl.program_id(0)
    xb = x_ref[...]
    score_t = jax.lax.dot_general(
        w_ref[...], xb, (((1,), (1,)), ((), ())),
        preferred_element_type=jnp.float32) + bias_ref[...][:, None]
    e_t = jnp.exp(score_t)
    e_ref[...] = e_t
    col_iota = jax.lax.broadcasted_iota(jnp.int32, (H, BN), 1)
    e_safe = jnp.where(col_iota + i * BN < N, e_t, 0.0)
    bblk = b3_ref[0, 0, :]
    seg_iota = jax.lax.broadcasted_iota(jnp.int32, (BN, S), 1)
    onehot = (seg_iota == bblk[:, None]).astype(jnp.float32)
    contrib = jax.lax.dot_general(
        e_safe, onehot, (((1,), (0,)), ((), ())),
        preferred_element_type=jnp.float32)

    @pl.when(i == 0)
    def _():
        denom_ref[...] = jnp.zeros_like(denom_ref)

    denom_ref[...] += contrib

    @pl.when(i == NB - 1)
    def _():
        rd_ref[...] = 1.0 / (denom_ref[...] + 1e-16)


def _sc_pool_body(x_hbm, e_hbm, rd_hbm, batch_hbm,
                  scoresm_hbm, out_hbm,
                  xb, ib, it, sbuf, smbuf, wbuf, rd2, zb, acc):
    cid = lax.axis_index("c")
    sid = lax.axis_index("s")
    wid = sid * NC + cid

    def zrow(r, carry):
        for c in range(D // L):
            zb[r, pl.ds(L * c, L)] = jnp.zeros((L,), jnp.float32)
        return carry

    lax.fori_loop(0, S // NS, zrow, 0)
    pltpu.sync_copy(zb, acc.at[pl.ds(sid * (S // NS), S // NS), :])

    pltpu.sync_copy(rd_hbm, rd2)
    plsc.subcore_barrier()

    iota = lax.iota(jnp.int32, L)
    iota_h8 = iota * H

    def process(nrows, base, idx_ref):
        def gbody(g, carry):
            bvec = idx_ref[pl.ds(g * L, L)]
            rvec = g * L + iota
            nidx = g * (L * H) + iota_h8
            w16 = jnp.zeros((L,), jnp.float32)
            for h in range(H):
                hvec = jnp.full((L,), h, jnp.int32)
                e = plsc.load_gather(sbuf, [hvec, rvec])
                rdv = plsc.load_gather(rd2, [hvec, bvec])
                sm = e * rdv
                plsc.store_scatter(smbuf, [nidx + h], sm)
                w16 = w16 + sm
            wbuf[pl.ds(g * L, L)] = w16
            return carry

        lax.fori_loop(0, nrows // L, gbody, 0)
        pltpu.sync_copy(smbuf.at[pl.ds(0, nrows * H)],
                        scoresm_hbm.at[pl.ds(base * H, nrows * H)])

        def sbody(g, carry):
            wv = wbuf[pl.ds(g * L, L)]
            for j in range(L):
                s = wv[j]
                r = g * L + j
                for c in range(D // L):
                    xb[r, pl.ds(L * c, L)] = xb[r, pl.ds(L * c, L)] * s
            return carry

        lax.fori_loop(0, nrows // L, sbody, 0)

    nblk = NBLK_BASE + jnp.where(wid < EXTRA, 1, 0)

    def body(i, carry):
        blk = wid + NW * i
        base = blk * C
        pltpu.sync_copy(x_hbm.at[pl.ds(base, C), :], xb)
        pltpu.sync_copy(batch_hbm.at[pl.ds(base, C)], ib)
        pltpu.sync_copy(e_hbm.at[:, pl.ds(base, C)], sbuf)
        process(C, base, ib)
        pltpu.sync_copy(xb, acc.at[ib], add=True)
        return carry

    lax.fori_loop(0, nblk, body, 0)

    @pl.when(wid == NW - 1)
    def _():
        base = NBLK * C
        pltpu.sync_copy(x_hbm.at[pl.ds(base, TAIL), :], xb.at[pl.ds(0, TAIL), :])
        pltpu.sync_copy(batch_hbm.at[pl.ds(base, TAIL)], it)
        pltpu.sync_copy(e_hbm.at[:, pl.ds(base, TAIL)],
                        sbuf.at[:, pl.ds(0, TAIL)])
        process(TAIL, base, it)
        pltpu.sync_copy(xb.at[pl.ds(0, TAIL), :], acc.at[it], add=True)

    plsc.subcore_barrier()

    @pl.when(sid == 0)
    def _():
        pltpu.sync_copy(acc, out_hbm.at[cid])


def _combine(p_ref, out_ref):
    out_ref[...] = p_ref[0] + p_ref[1]


def kernel(x, batch, W, b):
    batch_i32 = batch.astype(jnp.int32)
    batch3 = jnp.pad(batch_i32, (0, NP - N), constant_values=-1).reshape(
        NB, 1, BN)
    e_t, rd_t, _denom = pl.pallas_call(
        _pass_a,
        grid=(NB,),
        in_specs=[
            pl.BlockSpec((BN, D), lambda i: (i, 0)),
            pl.BlockSpec((1, 1, BN), lambda i: (i, 0, 0)),
            pl.BlockSpec((H, D), lambda i: (0, 0)),
            pl.BlockSpec((H,), lambda i: (0,)),
        ],
        out_specs=[
            pl.BlockSpec((H, BN), lambda i: (0, i)),
            pl.BlockSpec((H, S), lambda i: (0, 0)),
            pl.BlockSpec((H, S), lambda i: (0, 0)),
        ],
        out_shape=[
            jax.ShapeDtypeStruct((H, NP), jnp.float32),
            jax.ShapeDtypeStruct((H, S), jnp.float32),
            jax.ShapeDtypeStruct((H, S), jnp.float32),
        ],
    )(x, batch3, W, b)

    mesh = plsc.VectorSubcoreMesh(
        core_axis_name="c", subcore_axis_name="s",
        num_cores=NC, num_subcores=NS)
    scoresm_flat, partials = pl.kernel(
        _sc_pool_body,
        out_type=[
            jax.ShapeDtypeStruct((N * H,), jnp.float32),
            jax.ShapeDtypeStruct((NC, S, D), jnp.float32),
        ],
        mesh=mesh,
        compiler_params=pltpu.CompilerParams(needs_layout_passes=False),
        scratch_types=[
            pltpu.VMEM((C, D), jnp.float32),
            pltpu.VMEM((C,), jnp.int32),
            pltpu.VMEM((TAIL,), jnp.int32),
            pltpu.VMEM((H, C), jnp.float32),
            pltpu.VMEM((C * H,), jnp.float32),
            pltpu.VMEM((C,), jnp.float32),
            pltpu.VMEM((H, S), jnp.float32),
            pltpu.VMEM((S // NS, D), jnp.float32),
            pltpu.VMEM_SHARED((S, D), jnp.float32),
        ],
    )(x, e_t, rd_t, batch_i32)
    score_sm = scoresm_flat.reshape(N, H)

    value = pl.pallas_call(
        _combine,
        out_shape=jax.ShapeDtypeStruct((S, D), jnp.float32),
    )(partials)
    return (value, score_sm)


# speedup vs baseline: 1.0012x; 1.0012x over previous
import jax
import jax.numpy as jnp
from jax import lax
from jax.experimental import pallas as pl
from jax.experimental.pallas import tpu as pltpu
from jax.experimental.pallas import tpu_sc as plsc

N = 100000
D = 128
H = 8
S = 512
BN = 4096
NB = -(-N // BN)
NP = NB * BN

NC = 2
NS = 16
NW = NC * NS
C = 128
NBLK = N // C
TAIL = N - NBLK * C
EXTRA = NBLK - (NBLK // NW) * NW
NBLK_BASE = NBLK // NW
L = 16


def _pass_a(x_ref, b3_ref, w_ref, bias_ref, e_ref, rd_ref, denom_ref):
    i = pl.program_id(0)
    xb = x_ref[...]
    score_t = jax.lax.dot_general(
        w_ref[...], xb, (((1,), (1,)), ((), ())),
        preferred_element_type=jnp.float32) + bias_ref[...][:, None]
    e_t = jnp.exp(score_t)
    e_ref[...] = e_t
    col_iota = jax.lax.broadcasted_iota(jnp.int32, (H, BN), 1)
    e_safe = jnp.where(col_iota + i * BN < N, e_t, 0.0)
    bblk = b3_ref[0, 0, :]
    seg_iota = jax.lax.broadcasted_iota(jnp.int32, (BN, S), 1)
    onehot = (seg_iota == bblk[:, None]).astype(jnp.float32)
    contrib = jax.lax.dot_general(
        e_safe, onehot, (((1,), (0,)), ((), ())),
        preferred_element_type=jnp.float32)

    @pl.when(i == 0)
    def _():
        denom_ref[...] = jnp.zeros_like(denom_ref)

    denom_ref[...] += contrib

    @pl.when(i == NB - 1)
    def _():
        rd_ref[...] = 1.0 / (denom_ref[...] + 1e-16)


def _sc_pool_body(x_hbm, e_hbm, rd_hbm, batch_hbm,
                  scoresm_hbm, out_hbm,
                  xb, ib, it, sbuf, smbuf, wbuf, rd2, zb, acc):
    cid = lax.axis_index("c")
    sid = lax.axis_index("s")
    wid = sid * NC + cid

    def zrow(r, carry):
        for c in range(D // L):
            zb[r, pl.ds(L * c, L)] = jnp.zeros((L,), jnp.float32)
        return carry

    lax.fori_loop(0, S // NS, zrow, 0)
    pltpu.sync_copy(zb, acc.at[pl.ds(sid * (S // NS), S // NS), :])

    pltpu.sync_copy(rd_hbm, rd2)
    plsc.subcore_barrier()

    iota = lax.iota(jnp.int32, L)
    iota_h8 = iota * H

    def process(nrows, base, idx_ref, sbuf, xb):
        def gbody(g, carry):
            bvec = idx_ref[pl.ds(g * L, L)]
            rvec = g * L + iota
            nidx = g * (L * H) + iota_h8
            w16 = jnp.zeros((L,), jnp.float32)
            for h in range(H):
                hvec = jnp.full((L,), h, jnp.int32)
                e = plsc.load_gather(sbuf, [hvec, rvec])
                rdv = plsc.load_gather(rd2, [hvec, bvec])
                sm = e * rdv
                plsc.store_scatter(smbuf, [nidx + h], sm)
                w16 = w16 + sm
            wbuf[pl.ds(g * L, L)] = w16
            return carry

        lax.fori_loop(0, nrows // L, gbody, 0)
        pltpu.sync_copy(smbuf.at[pl.ds(0, nrows * H)],
                        scoresm_hbm.at[pl.ds(base * H, nrows * H)])

        def sbody(g, carry):
            wv = wbuf[pl.ds(g * L, L)]
            for j in range(L):
                s = wv[j]
                r = g * L + j
                for c in range(D // L):
                    xb[r, pl.ds(L * c, L)] = xb[r, pl.ds(L * c, L)] * s
            return carry

        lax.fori_loop(0, nrows // L, sbody, 0)

    nblk = NBLK_BASE + jnp.where(wid < EXTRA, 1, 0)

    def body(i, carry):
        blk = wid + NW * i
        base = blk * C
        pltpu.sync_copy(x_hbm.at[pl.ds(base, C), :], xb)
        pltpu.sync_copy(batch_hbm.at[pl.ds(base, C)], ib)
        pltpu.sync_copy(e_hbm.at[:, pl.ds(base, C)], sbuf)
        process(C, base, ib, sbuf, xb)
        pltpu.sync_copy(xb, acc.at[ib], add=True)
        return carry

    lax.fori_loop(0, nblk, body, 0)

    @pl.when(wid == NW - 1)
    def _():
        base = NBLK * C
        pltpu.sync_copy(x_hbm.at[pl.ds(base, TAIL), :], xb.at[pl.ds(0, TAIL), :])
        pltpu.sync_copy(batch_hbm.at[pl.ds(base, TAIL)], it)
        pltpu.sync_copy(e_hbm.at[:, pl.ds(base, TAIL)],
                        sbuf.at[:, pl.ds(0, TAIL)])
        process(TAIL, base, it, sbuf, xb)
        pltpu.sync_copy(xb.at[pl.ds(0, TAIL), :], acc.at[it], add=True)

    plsc.subcore_barrier()

    @pl.when(sid == 0)
    def _():
        pltpu.sync_copy(acc, out_hbm.at[cid])


def _combine(p_ref, out_ref):
    out_ref[...] = p_ref[0] + p_ref[1]


def kernel(x, batch, W, b):
    batch_i32 = batch.astype(jnp.int32)
    batch3 = jnp.pad(batch_i32, (0, NP - N), constant_values=-1).reshape(
        NB, 1, BN)
    e_t, rd_t, _denom = pl.pallas_call(
        _pass_a,
        grid=(NB,),
        in_specs=[
            pl.BlockSpec((BN, D), lambda i: (i, 0)),
            pl.BlockSpec((1, 1, BN), lambda i: (i, 0, 0)),
            pl.BlockSpec((H, D), lambda i: (0, 0)),
            pl.BlockSpec((H,), lambda i: (0,)),
        ],
        out_specs=[
            pl.BlockSpec((H, BN), lambda i: (0, i)),
            pl.BlockSpec((H, S), lambda i: (0, 0)),
            pl.BlockSpec((H, S), lambda i: (0, 0)),
        ],
        out_shape=[
            jax.ShapeDtypeStruct((H, NP), jnp.float32),
            jax.ShapeDtypeStruct((H, S), jnp.float32),
            jax.ShapeDtypeStruct((H, S), jnp.float32),
        ],
    )(x, batch3, W, b)

    mesh = plsc.VectorSubcoreMesh(
        core_axis_name="c", subcore_axis_name="s",
        num_cores=NC, num_subcores=NS)
    scoresm_flat, partials = pl.kernel(
        _sc_pool_body,
        out_type=[
            jax.ShapeDtypeStruct((N * H,), jnp.float32),
            jax.ShapeDtypeStruct((NC, S, D), jnp.float32),
        ],
        mesh=mesh,
        compiler_params=pltpu.CompilerParams(needs_layout_passes=False),
        scratch_types=[
            pltpu.VMEM((C, D), jnp.float32),
            pltpu.VMEM((C,), jnp.int32),
            pltpu.VMEM((TAIL,), jnp.int32),
            pltpu.VMEM((H, C), jnp.float32),
            pltpu.VMEM((C * H,), jnp.float32),
            pltpu.VMEM((C,), jnp.float32),
            pltpu.VMEM((H, S), jnp.float32),
            pltpu.VMEM((S // NS, D), jnp.float32),
            pltpu.VMEM_SHARED((S, D), jnp.float32),
        ],
    )(x, e_t, rd_t, batch_i32)
    score_sm = scoresm_flat.reshape(N, H)

    value = pl.pallas_call(
        _combine,
        out_shape=jax.ShapeDtypeStruct((S, D), jnp.float32),
    )(partials)
    return (value, score_sm)
